# baseline (speedup 1.0000x reference)
import functools

import jax
import jax.numpy as jnp
from jax import lax
from jax.experimental import pallas as pl
from jax.experimental.pallas import tpu as pltpu
from jax.experimental.pallas import tpu_sc as plsc

N = 10000
E = 320000
F_IN = 128
H_DIM = 64
N_CLASSES = 64
ALPHA = 0.1
NITER = 10

NC = 2
NS = 16
COLH = N_CLASSES // NC
EPT = E // NS
CHUNK = 128
NCH = 158
EPT_PAD = NCH * CHUNK
NPAD = 10240
NPT = NPAD // NS


def _mlp_body(a_ref, w1_ref, w2_ref, o_ref):
    h = jnp.maximum(
        jnp.dot(a_ref[...], w1_ref[...], preferred_element_type=jnp.float32),
        0.0,
    )
    o_ref[...] = jnp.dot(h, w2_ref[...], preferred_element_type=jnp.float32)


def _local_logits(attr_matrix, W1, W2):
    blk = 400
    return pl.pallas_call(
        _mlp_body,
        grid=(N // blk,),
        in_specs=[
            pl.BlockSpec((blk, F_IN), lambda i: (i, 0)),
            pl.BlockSpec((F_IN, H_DIM), lambda i: (0, 0)),
            pl.BlockSpec((H_DIM, N_CLASSES), lambda i: (0, 0)),
        ],
        out_specs=pl.BlockSpec((blk, N_CLASSES), lambda i: (i, 0)),
        out_shape=jax.ShapeDtypeStruct((N, N_CLASSES), jnp.float32),
    )(attr_matrix, W1, W2)


def _propagate(logits2, srcp, dstp, wp):
    mesh = plsc.VectorSubcoreMesh(
        core_axis_name="c", subcore_axis_name="s", num_cores=NC, num_subcores=NS
    )

    def body(logits_hbm, src_hbm, dst_hbm, w_hbm, z_hbm,
             idx_v, dst_v, w_v, logit_v, stage_v, rows_v, agg_sh,
             sem0, sem1):
        c = lax.axis_index("c")
        s = lax.axis_index("s")
        row0 = s * NPT
        sems = (sem0, sem1)

        pltpu.sync_copy(src_hbm.at[s], idx_v)
        pltpu.sync_copy(dst_hbm.at[s], dst_v)
        pltpu.sync_copy(w_hbm.at[s], w_v)
        pltpu.sync_copy(logits_hbm.at[c, pl.ds(row0, NPT), :], logit_v)
        pltpu.sync_copy(logit_v, z_hbm.at[c, pl.ds(row0, NPT), :])

        def zero_stage(r, carry):
            z16 = jnp.zeros((16,), jnp.float32)
            stage_v[r, pl.ds(0, 16)] = z16
            stage_v[r, pl.ds(16, 16)] = z16
            return carry

        def scale_rows(buf, j):
            for g in range(8):
                wg = w_v[j, g, :]
                for k in range(16):
                    e = g * 16 + k
                    w_s = wg[k]
                    buf[e, pl.ds(0, 16)] = buf[e, pl.ds(0, 16)] * w_s
                    buf[e, pl.ds(16, 16)] = buf[e, pl.ds(16, 16)] * w_s

        def edge_pair(j0, carry):
            for b in range(2):
                j = j0 * 2 + b
                buf = rows_v.at[b]
                pltpu.make_async_copy(
                    z_hbm.at[c].at[idx_v.at[j]], buf, sems[b]
                ).wait()
                scale_rows(buf, j)
                pltpu.sync_copy(buf, agg_sh.at[dst_v.at[j]], add=True)

                @pl.when(j < NCH - 2)
                def _():
                    pltpu.async_copy(
                        z_hbm.at[c].at[idx_v.at[j + 2]], buf, sems[b]
                    )
            return carry

        def update(r, carry):
            for h in range(2):
                sl = pl.ds(h * 16, 16)
                stage_v[r, sl] = (
                    (1.0 - ALPHA) * stage_v[r, sl] + ALPHA * logit_v[r, sl]
                )
            return carry

        def one_iter(it, carry):
            lax.fori_loop(0, NPT, zero_stage, 0, unroll=4)
            pltpu.sync_copy(stage_v, agg_sh.at[pl.ds(row0, NPT), :])
            plsc.subcore_barrier()
            pltpu.async_copy(z_hbm.at[c].at[idx_v.at[0]], rows_v.at[0], sem0)
            pltpu.async_copy(z_hbm.at[c].at[idx_v.at[1]], rows_v.at[1], sem1)
            lax.fori_loop(0, NCH // 2, edge_pair, 0)
            plsc.subcore_barrier()
            pltpu.sync_copy(agg_sh.at[pl.ds(row0, NPT), :], stage_v)
            lax.fori_loop(0, NPT, update, 0, unroll=4)
            pltpu.sync_copy(stage_v, z_hbm.at[c, pl.ds(row0, NPT), :])
            plsc.subcore_barrier()
            return carry

        plsc.subcore_barrier()
        lax.fori_loop(0, NITER, one_iter, 0)

    fn = pl.kernel(
        body,
        out_type=jax.ShapeDtypeStruct((NC, NPAD, COLH), jnp.float32),
        mesh=mesh,
        compiler_params=pltpu.CompilerParams(use_tc_tiling_on_sc=False),
        scratch_types=[
            pltpu.VMEM((NCH, CHUNK), jnp.int32),
            pltpu.VMEM((NCH, CHUNK), jnp.int32),
            pltpu.VMEM((NCH, 8, 16), jnp.float32),
            pltpu.VMEM((NPT, COLH), jnp.float32),
            pltpu.VMEM((NPT, COLH), jnp.float32),
            pltpu.VMEM((2, CHUNK, COLH), jnp.float32),
            pltpu.VMEM_SHARED((NPAD, COLH), jnp.float32),
            pltpu.SemaphoreType.DMA,
            pltpu.SemaphoreType.DMA,
        ],
    )
    return fn(logits2, srcp, dstp, wp)


def kernel(attr_matrix, edge_index, edge_weight, W1, W2):
    logits = _local_logits(attr_matrix, W1, W2)
    logits_p = jnp.pad(logits, ((0, NPAD - N), (0, 0)))
    logits2 = jnp.stack([logits_p[:, :COLH], logits_p[:, COLH:]], axis=0)

    src = edge_index[0].reshape(NS, EPT)
    dst = edge_index[1].reshape(NS, EPT)
    w = edge_weight.reshape(NS, EPT)
    pad = EPT_PAD - EPT
    srcp = jnp.pad(src, ((0, 0), (0, pad))).reshape(NS, NCH, CHUNK)
    dstp = jnp.pad(dst, ((0, 0), (0, pad))).reshape(NS, NCH, CHUNK)
    wp = jnp.pad(w, ((0, 0), (0, pad))).reshape(NS, NCH, 8, 16)

    z2 = _propagate(logits2, srcp, dstp, wp)
    return jnp.concatenate([z2[0, :N], z2[1, :N]], axis=1)


# speedup vs baseline: 1.6127x; 1.6127x over previous
import functools

import jax
import jax.numpy as jnp
from jax import lax
from jax.experimental import pallas as pl
from jax.experimental.pallas import tpu as pltpu
from jax.experimental.pallas import tpu_sc as plsc

N = 10000
E = 320000
F_IN = 128
H_DIM = 64
N_CLASSES = 64
ALPHA = 0.1
NITER = 10

NC = 2
NS = 16
COLH = N_CLASSES // NC
EPT = E // NS
CHUNK = 128
NCH = 158
EPT_PAD = NCH * CHUNK
NPAD = 10240
NPT = NPAD // NS


def _mlp_body(a_ref, w1_ref, w2_ref, o_ref):
    h = jnp.maximum(
        jnp.dot(a_ref[...], w1_ref[...], preferred_element_type=jnp.float32),
        0.0,
    )
    o_ref[...] = jnp.dot(h, w2_ref[...], preferred_element_type=jnp.float32)


def _local_logits(attr_matrix, W1, W2):
    blk = 400
    return pl.pallas_call(
        _mlp_body,
        grid=(N // blk,),
        in_specs=[
            pl.BlockSpec((blk, F_IN), lambda i: (i, 0)),
            pl.BlockSpec((F_IN, H_DIM), lambda i: (0, 0)),
            pl.BlockSpec((H_DIM, N_CLASSES), lambda i: (0, 0)),
        ],
        out_specs=pl.BlockSpec((blk, N_CLASSES), lambda i: (i, 0)),
        out_shape=jax.ShapeDtypeStruct((N, N_CLASSES), jnp.float32),
    )(attr_matrix, W1, W2)


def _propagate(logits2, srcp, dstp, wp):
    mesh = plsc.VectorSubcoreMesh(
        core_axis_name="c", subcore_axis_name="s", num_cores=NC, num_subcores=NS
    )

    def body(logits_hbm, src_hbm, dst_hbm, w_hbm, z_hbm,
             idx_v, dst_v, w_v, rows_v, msgs_v, agg_sh, z_sh,
             sem0, sem1, ssem0, ssem1):
        c = lax.axis_index("c")
        s = lax.axis_index("s")
        row0 = s * NPT
        sems = (sem0, sem1)
        ssems = (ssem0, ssem1)
        NU = NPT // CHUNK

        pltpu.sync_copy(src_hbm.at[s], idx_v)
        pltpu.sync_copy(dst_hbm.at[s], dst_v)
        pltpu.sync_copy(w_hbm.at[s], w_v)
        pltpu.sync_copy(
            logits_hbm.at[c, pl.ds(row0, NPT), :], z_sh.at[pl.ds(row0, NPT), :]
        )

        def scale_rows(rbuf, mbuf, j):
            for g in range(8):
                wg = w_v[j, g, :]
                for k in range(16):
                    e = g * 16 + k
                    w_s = wg[k]
                    mbuf[e, pl.ds(0, 16)] = rbuf[e, pl.ds(0, 16)] * w_s
                    mbuf[e, pl.ds(16, 16)] = rbuf[e, pl.ds(16, 16)] * w_s

        def edge_pair(j0, carry):
            for b in range(2):
                j = j0 * 2 + b
                rbuf = rows_v.at[b]
                mbuf = msgs_v.at[b]
                pltpu.make_async_copy(
                    z_sh.at[idx_v.at[j]], rbuf, sems[b]
                ).wait()

                @pl.when(j >= 2)
                def _():
                    pltpu.make_async_copy(
                        mbuf, agg_sh.at[dst_v.at[j]], ssems[b]
                    ).wait()

                scale_rows(rbuf, mbuf, j)
                pltpu.async_copy(
                    mbuf, agg_sh.at[dst_v.at[j]], ssems[b], add=True
                )

                @pl.when(j < NCH - 2)
                def _():
                    pltpu.async_copy(
                        z_sh.at[idx_v.at[j + 2]], rbuf, sems[b]
                    )
            return carry

        def one_iter(it, carry):
            def zfill(r, carry2):
                z16 = jnp.zeros((16,), jnp.float32)
                msgs_v[1, r, pl.ds(0, 16)] = z16
                msgs_v[1, r, pl.ds(16, 16)] = z16
                return carry2

            lax.fori_loop(0, CHUNK, zfill, 0, unroll=8)
            for u in range(NU):
                pltpu.sync_copy(
                    msgs_v.at[1],
                    agg_sh.at[pl.ds(row0 + u * CHUNK, CHUNK), :],
                )
            plsc.subcore_barrier()
            pltpu.async_copy(z_sh.at[idx_v.at[0]], rows_v.at[0], sem0)
            pltpu.async_copy(z_sh.at[idx_v.at[1]], rows_v.at[1], sem1)
            lax.fori_loop(0, NCH // 2, edge_pair, 0)
            for b in range(2):
                pltpu.make_async_copy(
                    msgs_v.at[b], agg_sh.at[dst_v.at[NCH - 2 + b]], ssems[b]
                ).wait()
            plsc.subcore_barrier()
            for u in range(NU):
                rsl = pl.ds(row0 + u * CHUNK, CHUNK)
                pltpu.sync_copy(agg_sh.at[rsl, :], rows_v.at[0])
                pltpu.sync_copy(logits_hbm.at[c, rsl, :], rows_v.at[1])

                def blend(r, carry2):
                    for h in range(2):
                        sl = pl.ds(h * 16, 16)
                        msgs_v[0, r, sl] = (
                            (1.0 - ALPHA) * rows_v[0, r, sl]
                            + ALPHA * rows_v[1, r, sl]
                        )
                    return carry2

                lax.fori_loop(0, CHUNK, blend, 0, unroll=4)
                pltpu.sync_copy(msgs_v.at[0], z_sh.at[rsl, :])

                @pl.when(it == NITER - 1)
                def _():
                    pltpu.sync_copy(msgs_v.at[0], z_hbm.at[c, rsl, :])

            plsc.subcore_barrier()
            return carry

        plsc.subcore_barrier()
        lax.fori_loop(0, NITER, one_iter, 0)

    fn = pl.kernel(
        body,
        out_type=jax.ShapeDtypeStruct((NC, NPAD, COLH), jnp.float32),
        mesh=mesh,
        compiler_params=pltpu.CompilerParams(use_tc_tiling_on_sc=False),
        scratch_types=[
            pltpu.VMEM((NCH, CHUNK), jnp.int32),
            pltpu.VMEM((NCH, CHUNK), jnp.int32),
            pltpu.VMEM((NCH, 8, 16), jnp.float32),
            pltpu.VMEM((2, CHUNK, COLH), jnp.float32),
            pltpu.VMEM((2, CHUNK, COLH), jnp.float32),
            pltpu.VMEM_SHARED((NPAD, COLH), jnp.float32),
            pltpu.VMEM_SHARED((NPAD, COLH), jnp.float32),
            pltpu.SemaphoreType.DMA,
            pltpu.SemaphoreType.DMA,
            pltpu.SemaphoreType.DMA,
            pltpu.SemaphoreType.DMA,
        ],
    )
    return fn(logits2, srcp, dstp, wp)


def kernel(attr_matrix, edge_index, edge_weight, W1, W2):
    logits = _local_logits(attr_matrix, W1, W2)
    logits_p = jnp.pad(logits, ((0, NPAD - N), (0, 0)))
    logits2 = jnp.stack([logits_p[:, :COLH], logits_p[:, COLH:]], axis=0)

    src = edge_index[0].reshape(NS, EPT)
    dst = edge_index[1].reshape(NS, EPT)
    w = edge_weight.reshape(NS, EPT)
    pad = EPT_PAD - EPT
    srcp = jnp.pad(src, ((0, 0), (0, pad))).reshape(NS, NCH, CHUNK)
    dstp = jnp.pad(dst, ((0, 0), (0, pad))).reshape(NS, NCH, CHUNK)
    wp = jnp.pad(w, ((0, 0), (0, pad))).reshape(NS, NCH, 8, 16)

    z2 = _propagate(logits2, srcp, dstp, wp)
    return jnp.concatenate([z2[0, :N], z2[1, :N]], axis=1)
